# initial kernel scaffold (unmeasured)
import jax
import jax.numpy as jnp
from jax import lax
from jax.experimental import pallas as pl
from jax.experimental.pallas import tpu as pltpu


def kernel(
    x,
):
    def body(*refs):
        pass

    out_shape = jax.ShapeDtypeStruct(..., jnp.float32)
    return pl.pallas_call(body, out_shape=out_shape)(...)


# baseline (device time: 1204551 ns/iter reference)
import jax
import jax.numpy as jnp
from jax import lax
from jax.experimental import pallas as pl
from jax.experimental.pallas import tpu as pltpu

NZ = 4
P, M = 0, 1


def kernel(x):
    m, n = x.shape
    h = m // 2
    x16 = x.astype(jnp.bfloat16)

    def body(x_ref, out_ref, send_sems, recv_sems, copy_sem):
        my_x = lax.axis_index("x")
        my_y = lax.axis_index("y")
        my_z = lax.axis_index("z")
        right = lax.rem(my_z + 1, NZ)
        left = lax.rem(my_z + NZ - 1, NZ)

        barrier = pltpu.get_barrier_semaphore()
        for nbr in (left, right):
            pl.semaphore_signal(
                barrier, inc=1,
                device_id=(my_x, my_y, nbr),
                device_id_type=pl.DeviceIdType.MESH,
            )
        pl.semaphore_wait(barrier, 2)

        def top(o):
            return out_ref.at[pl.ds(o * m, h), :]

        def bot(o):
            return out_ref.at[pl.ds(o * m + h, h), :]

        def send(src, dst, d, hop, nbr):
            rdma = pltpu.make_async_remote_copy(
                src_ref=src, dst_ref=dst,
                send_sem=send_sems.at[d, hop], recv_sem=recv_sems.at[d, hop],
                device_id=(my_x, my_y, nbr),
                device_id_type=pl.DeviceIdType.MESH,
            )
            rdma.start()
            return rdma

        def wait_recv(dst, d, hop, nbr):
            rdma = pltpu.make_async_remote_copy(
                src_ref=dst, dst_ref=dst,
                send_sem=send_sems.at[d, hop], recv_sem=recv_sems.at[d, hop],
                device_id=(my_x, my_y, nbr),
                device_id_type=pl.DeviceIdType.MESH,
            )
            rdma.wait_recv()

        started = []
        started.append(send(x_ref.at[pl.ds(0, h), :], top(my_z), P, 0, right))
        started.append(send(x_ref.at[pl.ds(h, h), :], bot(my_z), M, 0, left))

        local = pltpu.make_async_copy(
            x_ref, out_ref.at[pl.ds(my_z * m, m), :], copy_sem
        )
        local.start()

        for hop in range(NZ - 1):
            o_p = lax.rem(my_z + NZ - 1 - hop, NZ)
            wait_recv(top(o_p), P, hop, right)
            if hop < NZ - 2:
                started.append(send(top(o_p), top(o_p), P, hop + 1, right))
            o_m = lax.rem(my_z + 1 + hop, NZ)
            wait_recv(bot(o_m), M, hop, left)
            if hop < NZ - 2:
                started.append(send(bot(o_m), bot(o_m), M, hop + 1, left))

        local.wait()
        for rdma in started:
            rdma.wait_send()

    return pl.pallas_call(
        body,
        out_shape=jax.ShapeDtypeStruct((NZ * m, n), jnp.bfloat16),
        in_specs=[pl.BlockSpec(memory_space=pl.ANY)],
        out_specs=pl.BlockSpec(memory_space=pl.ANY),
        scratch_shapes=[
            pltpu.SemaphoreType.DMA((2, NZ - 1)),
            pltpu.SemaphoreType.DMA((2, NZ - 1)),
            pltpu.SemaphoreType.DMA,
        ],
        compiler_params=pltpu.CompilerParams(collective_id=0),
    )(x16)


# device time: 1143651 ns/iter; 1.0533x vs baseline; 1.0533x over previous
import jax
import jax.numpy as jnp
from jax import lax
from jax.experimental import pallas as pl
from jax.experimental.pallas import tpu as pltpu

NZ = 4
CW, CCW = 0, 1
MESH = pl.DeviceIdType.MESH


def kernel(x):
    m, n = x.shape
    q = m // 4
    p = q // 2
    x16 = x.astype(jnp.bfloat16)

    def body(x_ref, out_ref, zsend, zrecv, psend, precv, copy_sem):
        X = lax.axis_index("x")
        Y = lax.axis_index("y")
        Z = lax.axis_index("z")
        r = 2 * X + jnp.bitwise_xor(X, Y)

        def mod4(v):
            return lax.rem(v + 8, NZ)

        def pos_coords(pos):
            pos = mod4(pos)
            px = pos // 2
            return px, jnp.bitwise_xor(lax.rem(pos, 2), px)

        cwx, cwy = pos_coords(r + 1)
        ccwx, ccwy = pos_coords(r - 1)
        cw_tgt = (cwx, cwy, Z)
        ccw_tgt = (ccwx, ccwy, Z)

        barrier = pltpu.get_barrier_semaphore()
        for tgt in (
            (X, Y, mod4(Z + 1)),
            (X, Y, mod4(Z - 1)),
            cw_tgt,
            ccw_tgt,
        ):
            pl.semaphore_signal(barrier, inc=1, device_id=tgt,
                                device_id_type=MESH)
        pl.semaphore_wait(barrier, 4)

        def rows(o, start, size):
            return out_ref.at[pl.ds(o * m + start, size), :]

        pending = []

        def send(src, dst, sem_s, sem_r, tgt, cond):
            rdma = pltpu.make_async_remote_copy(
                src_ref=src, dst_ref=dst, send_sem=sem_s, recv_sem=sem_r,
                device_id=tgt, device_id_type=MESH)

            @pl.when(cond)
            def _():
                rdma.start()

            pending.append((cond, rdma))

        def wait_recv(dst, sem_s, sem_r, cond):
            rdma = pltpu.make_async_remote_copy(
                src_ref=dst, dst_ref=dst, send_sem=sem_s, recv_sem=sem_r,
                device_id=(X, Y, Z), device_id_type=MESH)

            @pl.when(cond)
            def _():
                rdma.wait_recv()

        def slot(flavor, h):
            if flavor == "R":
                return dict(o=mod4(Z - 1 - h), cond=(Z >= h + 1), j=2 - h)
            return dict(o=mod4(Z + 1 + h), cond=(Z <= 2 - h), j=h)

        def plane_hop0(s):
            o, j, cond = s["o"], s["j"], s["cond"]
            send(rows(o, r * q, p), rows(o, r * q, p),
                 psend.at[j, CW, 0], precv.at[j, CW, 0], cw_tgt, cond)
            send(rows(o, r * q + p, p), rows(o, r * q + p, p),
                 psend.at[j, CCW, 0], precv.at[j, CCW, 0], ccw_tgt, cond)

        def plane_step(s, k):
            o, j, cond = s["o"], s["j"], s["cond"]
            rc = mod4(r - 1 - k)
            wait_recv(rows(o, rc * q, p),
                      psend.at[j, CW, k], precv.at[j, CW, k], cond)
            if k < 2:
                send(rows(o, rc * q, p), rows(o, rc * q, p),
                     psend.at[j, CW, k + 1], precv.at[j, CW, k + 1],
                     cw_tgt, cond)
            rl = mod4(r + 1 + k)
            wait_recv(rows(o, rl * q + p, p),
                      psend.at[j, CCW, k], precv.at[j, CCW, k], cond)
            if k < 2:
                send(rows(o, rl * q + p, p), rows(o, rl * q + p, p),
                     psend.at[j, CCW, k + 1], precv.at[j, CCW, k + 1],
                     ccw_tgt, cond)

        xq = x_ref.at[pl.ds(r * q, q), :]
        send(xq, rows(Z, r * q, q), zsend.at[0, 0], zrecv.at[0, 0],
             (X, Y, mod4(Z + 1)), Z <= 2)
        send(xq, rows(Z, r * q, q), zsend.at[1, 0], zrecv.at[1, 0],
             (X, Y, mod4(Z - 1)), Z >= 1)

        local = pltpu.make_async_copy(
            x_ref, out_ref.at[pl.ds(Z * m, m), :], copy_sem)
        local.start()

        for t in range(6):
            if t < 3:
                for flavor, znbr, f in (("R", mod4(Z + 1), 0),
                                        ("L", mod4(Z - 1), 1)):
                    s = slot(flavor, t)
                    wait_recv(rows(s["o"], r * q, q),
                              zsend.at[f, t], zrecv.at[f, t], s["cond"])
                    if t < 2:
                        fwd_ok = s["cond"] & (
                            (Z <= 2) if flavor == "R" else (Z >= 1))
                        send(rows(s["o"], r * q, q), rows(s["o"], r * q, q),
                             zsend.at[f, t + 1], zrecv.at[f, t + 1],
                             (X, Y, znbr), fwd_ok)
                    plane_hop0(s)
            for h in range(3):
                k = t - 1 - h
                if 0 <= k <= 2:
                    plane_step(slot("R", h), k)
                    plane_step(slot("L", h), k)

        local.wait()
        for cond, rdma in pending:
            @pl.when(cond)
            def _(rdma=rdma):
                rdma.wait_send()

    return pl.pallas_call(
        body,
        out_shape=jax.ShapeDtypeStruct((NZ * m, n), jnp.bfloat16),
        in_specs=[pl.BlockSpec(memory_space=pl.ANY)],
        out_specs=pl.BlockSpec(memory_space=pl.ANY),
        scratch_shapes=[
            pltpu.SemaphoreType.DMA((2, 3)),
            pltpu.SemaphoreType.DMA((2, 3)),
            pltpu.SemaphoreType.DMA((3, 2, 3)),
            pltpu.SemaphoreType.DMA((3, 2, 3)),
            pltpu.SemaphoreType.DMA,
        ],
        compiler_params=pltpu.CompilerParams(collective_id=0),
    )(x16)


# device time: 832788 ns/iter; 1.4464x vs baseline; 1.3733x over previous
import jax
import jax.numpy as jnp
from jax import lax
from jax.experimental import pallas as pl
from jax.experimental.pallas import tpu as pltpu

NZ = 4
CW, CCW = 0, 1
MESH = pl.DeviceIdType.MESH


def kernel(x):
    m, n = x.shape
    q = m // 4
    p = q // 2
    x16 = x.astype(jnp.bfloat16)

    def body(x_ref, out_ref, zsend, zrecv, psend, precv):
        X = lax.axis_index("x")
        Y = lax.axis_index("y")
        Z = lax.axis_index("z")
        r = 2 * X + jnp.bitwise_xor(X, Y)

        def mod4(v):
            return lax.rem(v + 8, NZ)

        def pos_coords(pos):
            pos = mod4(pos)
            px = pos // 2
            return px, jnp.bitwise_xor(lax.rem(pos, 2), px)

        cwx, cwy = pos_coords(r + 1)
        ccwx, ccwy = pos_coords(r - 1)
        cw_tgt = (cwx, cwy, Z)
        ccw_tgt = (ccwx, ccwy, Z)

        barrier = pltpu.get_barrier_semaphore()
        for tgt in (
            (X, Y, mod4(Z + 1)),
            (X, Y, mod4(Z - 1)),
            cw_tgt,
            ccw_tgt,
        ):
            pl.semaphore_signal(barrier, inc=1, device_id=tgt,
                                device_id_type=MESH)
        pl.semaphore_wait(barrier, 4)

        def rows(o, start, size):
            return out_ref.at[pl.ds(o * m + start, size), :]

        pending = []

        def send(src, dst, sem_s, sem_r, tgt, cond):
            rdma = pltpu.make_async_remote_copy(
                src_ref=src, dst_ref=dst, send_sem=sem_s, recv_sem=sem_r,
                device_id=tgt, device_id_type=MESH)

            @pl.when(cond)
            def _():
                rdma.start()

            pending.append((cond, rdma))

        def wait_recv(dst, sem_s, sem_r, cond):
            rdma = pltpu.make_async_remote_copy(
                src_ref=dst, dst_ref=dst, send_sem=sem_s, recv_sem=sem_r,
                device_id=(X, Y, Z), device_id_type=MESH)

            @pl.when(cond)
            def _():
                rdma.wait_recv()

        def slot(flavor, h):
            if flavor == "R":
                return dict(o=mod4(Z - 1 - h), cond=(Z >= h + 1), j=2 - h)
            return dict(o=mod4(Z + 1 + h), cond=(Z <= 2 - h), j=h)

        def plane_hop0(s):
            o, j, cond = s["o"], s["j"], s["cond"]
            send(rows(o, r * q, p), rows(o, r * q, p),
                 psend.at[j, CW, 0], precv.at[j, CW, 0], cw_tgt, cond)
            send(rows(o, r * q + p, p), rows(o, r * q + p, p),
                 psend.at[j, CCW, 0], precv.at[j, CCW, 0], ccw_tgt, cond)

        def plane_step(s, k):
            o, j, cond = s["o"], s["j"], s["cond"]
            rc = mod4(r - 1 - k)
            wait_recv(rows(o, rc * q, p),
                      psend.at[j, CW, k], precv.at[j, CW, k], cond)
            if k < 2:
                send(rows(o, rc * q, p), rows(o, rc * q, p),
                     psend.at[j, CW, k + 1], precv.at[j, CW, k + 1],
                     cw_tgt, cond)
            rl = mod4(r + 1 + k)
            wait_recv(rows(o, rl * q + p, p),
                      psend.at[j, CCW, k], precv.at[j, CCW, k], cond)
            if k < 2:
                send(rows(o, rl * q + p, p), rows(o, rl * q + p, p),
                     psend.at[j, CCW, k + 1], precv.at[j, CCW, k + 1],
                     ccw_tgt, cond)

        xq = x_ref.at[pl.ds(r * q, q), :]
        send(xq, rows(Z, r * q, q), zsend.at[0, 0], zrecv.at[0, 0],
             (X, Y, mod4(Z + 1)), Z <= 2)
        send(xq, rows(Z, r * q, q), zsend.at[1, 0], zrecv.at[1, 0],
             (X, Y, mod4(Z - 1)), Z >= 1)

        for t in range(6):
            if t < 3:
                for flavor, znbr, f in (("R", mod4(Z + 1), 0),
                                        ("L", mod4(Z - 1), 1)):
                    s = slot(flavor, t)
                    wait_recv(rows(s["o"], r * q, q),
                              zsend.at[f, t], zrecv.at[f, t], s["cond"])
                    if t < 2:
                        fwd_ok = s["cond"] & (
                            (Z <= 2) if flavor == "R" else (Z >= 1))
                        send(rows(s["o"], r * q, q), rows(s["o"], r * q, q),
                             zsend.at[f, t + 1], zrecv.at[f, t + 1],
                             (X, Y, znbr), fwd_ok)
                    plane_hop0(s)
            for h in range(3):
                k = t - 1 - h
                if 0 <= k <= 2:
                    plane_step(slot("R", h), k)
                    plane_step(slot("L", h), k)

        for cond, rdma in pending:
            @pl.when(cond)
            def _(rdma=rdma):
                rdma.wait_send()

    gathered = pl.pallas_call(
        body,
        out_shape=jax.ShapeDtypeStruct((NZ * m, n), jnp.bfloat16),
        in_specs=[pl.BlockSpec(memory_space=pl.ANY)],
        out_specs=pl.BlockSpec(memory_space=pl.ANY),
        scratch_shapes=[
            pltpu.SemaphoreType.DMA((2, 3)),
            pltpu.SemaphoreType.DMA((2, 3)),
            pltpu.SemaphoreType.DMA((3, 2, 3)),
            pltpu.SemaphoreType.DMA((3, 2, 3)),
        ],
        compiler_params=pltpu.CompilerParams(collective_id=0),
    )(x16)
    z = lax.axis_index("z")
    return lax.dynamic_update_slice(gathered, x16, (z * m, 0))


# device time: 733672 ns/iter; 1.6418x vs baseline; 1.1351x over previous
import jax
import jax.numpy as jnp
from jax import lax
from jax.experimental import pallas as pl
from jax.experimental.pallas import tpu as pltpu

NZ = 4
CW, CCW = 0, 1
MESH = pl.DeviceIdType.MESH


def kernel(x):
    m, n = x.shape
    q = m // 4
    p = q // 2
    x16 = x.astype(jnp.bfloat16)

    def body(x_ref, init_ref, out_ref, zsend, zrecv, psend, precv):
        X = lax.axis_index("x")
        Y = lax.axis_index("y")
        Z = lax.axis_index("z")
        r = 2 * X + jnp.bitwise_xor(X, Y)

        def mod4(v):
            return lax.rem(v + 8, NZ)

        def pos_coords(pos):
            pos = mod4(pos)
            px = pos // 2
            return px, jnp.bitwise_xor(lax.rem(pos, 2), px)

        cwx, cwy = pos_coords(r + 1)
        ccwx, ccwy = pos_coords(r - 1)
        cw_tgt = (cwx, cwy, Z)
        ccw_tgt = (ccwx, ccwy, Z)

        barrier = pltpu.get_barrier_semaphore()
        for tgt in (
            (X, Y, mod4(Z + 1)),
            (X, Y, mod4(Z - 1)),
            cw_tgt,
            ccw_tgt,
        ):
            pl.semaphore_signal(barrier, inc=1, device_id=tgt,
                                device_id_type=MESH)
        pl.semaphore_wait(barrier, 4)

        def rows(o, start, size):
            return out_ref.at[pl.ds(o * m + start, size), :]

        pending = []

        def send(src, dst, sem_s, sem_r, tgt, cond):
            rdma = pltpu.make_async_remote_copy(
                src_ref=src, dst_ref=dst, send_sem=sem_s, recv_sem=sem_r,
                device_id=tgt, device_id_type=MESH)

            @pl.when(cond)
            def _():
                rdma.start()

            pending.append((cond, rdma))

        def wait_recv(dst, sem_s, sem_r, cond):
            rdma = pltpu.make_async_remote_copy(
                src_ref=dst, dst_ref=dst, send_sem=sem_s, recv_sem=sem_r,
                device_id=(X, Y, Z), device_id_type=MESH)

            @pl.when(cond)
            def _():
                rdma.wait_recv()

        def slot(flavor, h):
            if flavor == "R":
                return dict(o=mod4(Z - 1 - h), cond=(Z >= h + 1), j=2 - h)
            return dict(o=mod4(Z + 1 + h), cond=(Z <= 2 - h), j=h)

        def plane_hop0(s):
            o, j, cond = s["o"], s["j"], s["cond"]
            send(rows(o, r * q, p), rows(o, r * q, p),
                 psend.at[j, CW, 0], precv.at[j, CW, 0], cw_tgt, cond)
            send(rows(o, r * q + p, p), rows(o, r * q + p, p),
                 psend.at[j, CCW, 0], precv.at[j, CCW, 0], ccw_tgt, cond)

        def plane_step(s, k):
            o, j, cond = s["o"], s["j"], s["cond"]
            rc = mod4(r - 1 - k)
            wait_recv(rows(o, rc * q, p),
                      psend.at[j, CW, k], precv.at[j, CW, k], cond)
            if k < 2:
                send(rows(o, rc * q, p), rows(o, rc * q, p),
                     psend.at[j, CW, k + 1], precv.at[j, CW, k + 1],
                     cw_tgt, cond)
            rl = mod4(r + 1 + k)
            wait_recv(rows(o, rl * q + p, p),
                      psend.at[j, CCW, k], precv.at[j, CCW, k], cond)
            if k < 2:
                send(rows(o, rl * q + p, p), rows(o, rl * q + p, p),
                     psend.at[j, CCW, k + 1], precv.at[j, CCW, k + 1],
                     ccw_tgt, cond)

        xq = x_ref.at[pl.ds(r * q, q), :]
        send(xq, rows(Z, r * q, q), zsend.at[0, 0], zrecv.at[0, 0],
             (X, Y, mod4(Z + 1)), Z <= 2)
        send(xq, rows(Z, r * q, q), zsend.at[1, 0], zrecv.at[1, 0],
             (X, Y, mod4(Z - 1)), Z >= 1)

        for t in range(6):
            if t < 3:
                for flavor, znbr, f in (("R", mod4(Z + 1), 0),
                                        ("L", mod4(Z - 1), 1)):
                    s = slot(flavor, t)
                    wait_recv(rows(s["o"], r * q, q),
                              zsend.at[f, t], zrecv.at[f, t], s["cond"])
                    if t < 2:
                        fwd_ok = s["cond"] & (
                            (Z <= 2) if flavor == "R" else (Z >= 1))
                        send(rows(s["o"], r * q, q), rows(s["o"], r * q, q),
                             zsend.at[f, t + 1], zrecv.at[f, t + 1],
                             (X, Y, znbr), fwd_ok)
                    plane_hop0(s)
            for h in range(3):
                k = t - 1 - h
                if 0 <= k <= 2:
                    plane_step(slot("R", h), k)
                    plane_step(slot("L", h), k)

        for cond, rdma in pending:
            @pl.when(cond)
            def _(rdma=rdma):
                rdma.wait_send()

    out_init = jnp.tile(x16, (NZ, 1))
    return pl.pallas_call(
        body,
        out_shape=jax.ShapeDtypeStruct((NZ * m, n), jnp.bfloat16),
        in_specs=[pl.BlockSpec(memory_space=pl.ANY),
                  pl.BlockSpec(memory_space=pl.ANY)],
        out_specs=pl.BlockSpec(memory_space=pl.ANY),
        input_output_aliases={1: 0},
        scratch_shapes=[
            pltpu.SemaphoreType.DMA((2, 3)),
            pltpu.SemaphoreType.DMA((2, 3)),
            pltpu.SemaphoreType.DMA((3, 2, 3)),
            pltpu.SemaphoreType.DMA((3, 2, 3)),
        ],
        compiler_params=pltpu.CompilerParams(collective_id=0),
    )(x16, out_init)


# device time: 732224 ns/iter; 1.6451x vs baseline; 1.0020x over previous
import jax
import jax.numpy as jnp
from jax import lax
from jax.experimental import pallas as pl
from jax.experimental.pallas import tpu as pltpu

NZ = 4
CW, CCW = 0, 1
MESH = pl.DeviceIdType.MESH


def kernel(x):
    m, n = x.shape
    q = m // 4
    p = q // 2

    def body(init_ref, out_ref, zsend, zrecv, psend, precv):
        X = lax.axis_index("x")
        Y = lax.axis_index("y")
        Z = lax.axis_index("z")
        r = 2 * X + jnp.bitwise_xor(X, Y)

        def mod4(v):
            return lax.rem(v + 8, NZ)

        def pos_coords(pos):
            pos = mod4(pos)
            px = pos // 2
            return px, jnp.bitwise_xor(lax.rem(pos, 2), px)

        cwx, cwy = pos_coords(r + 1)
        ccwx, ccwy = pos_coords(r - 1)
        cw_tgt = (cwx, cwy, Z)
        ccw_tgt = (ccwx, ccwy, Z)
        zup = (X, Y, mod4(Z + 1))
        zdn = (X, Y, mod4(Z - 1))

        barrier = pltpu.get_barrier_semaphore()
        for tgt in (zup, zdn, cw_tgt, ccw_tgt):
            pl.semaphore_signal(barrier, inc=1, device_id=tgt,
                                device_id_type=MESH)
        pl.semaphore_wait(barrier, 4)

        def rows(o, start, size):
            return out_ref.at[pl.ds(o * m + start, size), :]

        pending = []

        def send(src, dst, sem_s, sem_r, tgt, cond):
            rdma = pltpu.make_async_remote_copy(
                src_ref=src, dst_ref=dst, send_sem=sem_s, recv_sem=sem_r,
                device_id=tgt, device_id_type=MESH)

            @pl.when(cond)
            def _():
                rdma.start()

            pending.append((cond, rdma))

        def wait_recv(dst, sem_s, sem_r, cond):
            rdma = pltpu.make_async_remote_copy(
                src_ref=dst, dst_ref=dst, send_sem=sem_s, recv_sem=sem_r,
                device_id=(X, Y, Z), device_id_type=MESH)

            @pl.when(cond)
            def _():
                rdma.wait_recv()

        def slot(flavor, h):
            if flavor == "R":
                return dict(o=mod4(Z - 1 - h), cond=(Z >= h + 1), j=2 - h)
            return dict(o=mod4(Z + 1 + h), cond=(Z <= 2 - h), j=h)

        def plane_hop0(s, half):
            o, j, cond = s["o"], s["j"], s["cond"]
            if half == 0:
                send(rows(o, r * q, p), rows(o, r * q, p),
                     psend.at[j, CW, 0], precv.at[j, CW, 0], cw_tgt, cond)
            else:
                send(rows(o, r * q + p, p), rows(o, r * q + p, p),
                     psend.at[j, CCW, 0], precv.at[j, CCW, 0], ccw_tgt,
                     cond)

        def plane_step(s, k):
            o, j, cond = s["o"], s["j"], s["cond"]
            rc = mod4(r - 1 - k)
            wait_recv(rows(o, rc * q, p),
                      psend.at[j, CW, k], precv.at[j, CW, k], cond)
            if k < 2:
                send(rows(o, rc * q, p), rows(o, rc * q, p),
                     psend.at[j, CW, k + 1], precv.at[j, CW, k + 1],
                     cw_tgt, cond)
            rl = mod4(r + 1 + k)
            wait_recv(rows(o, rl * q + p, p),
                      psend.at[j, CCW, k], precv.at[j, CCW, k], cond)
            if k < 2:
                send(rows(o, rl * q + p, p), rows(o, rl * q + p, p),
                     psend.at[j, CCW, k + 1], precv.at[j, CCW, k + 1],
                     ccw_tgt, cond)

        for half in (0, 1):
            src = rows(Z, r * q + half * p, p)
            send(src, src, zsend.at[0, 0, half], zrecv.at[0, 0, half],
                 zup, Z <= 2)
            send(src, src, zsend.at[1, 0, half], zrecv.at[1, 0, half],
                 zdn, Z >= 1)

        for t in range(6):
            if t < 3:
                for flavor, ztgt, f in (("R", zup, 0), ("L", zdn, 1)):
                    s = slot(flavor, t)
                    fwd_ok = s["cond"] & (
                        (Z <= 2) if flavor == "R" else (Z >= 1))
                    for half in (0, 1):
                        piece = rows(s["o"], r * q + half * p, p)
                        wait_recv(piece, zsend.at[f, t, half],
                                  zrecv.at[f, t, half], s["cond"])
                        if t < 2:
                            send(piece, piece,
                                 zsend.at[f, t + 1, half],
                                 zrecv.at[f, t + 1, half], ztgt, fwd_ok)
                        plane_hop0(s, half)
            for h in range(3):
                k = t - 1 - h
                if 0 <= k <= 2:
                    plane_step(slot("R", h), k)
                    plane_step(slot("L", h), k)

        for cond, rdma in pending:
            @pl.when(cond)
            def _(rdma=rdma):
                rdma.wait_send()

    out_init = jnp.tile(x.astype(jnp.bfloat16), (NZ, 1))
    return pl.pallas_call(
        body,
        out_shape=jax.ShapeDtypeStruct((NZ * m, n), jnp.bfloat16),
        in_specs=[pl.BlockSpec(memory_space=pl.ANY)],
        out_specs=pl.BlockSpec(memory_space=pl.ANY),
        input_output_aliases={0: 0},
        scratch_shapes=[
            pltpu.SemaphoreType.DMA((2, 3, 2)),
            pltpu.SemaphoreType.DMA((2, 3, 2)),
            pltpu.SemaphoreType.DMA((3, 2, 3)),
            pltpu.SemaphoreType.DMA((3, 2, 3)),
        ],
        compiler_params=pltpu.CompilerParams(collective_id=0),
    )(out_init)


# device time: 685798 ns/iter; 1.7564x vs baseline; 1.0677x over previous
import jax
import jax.numpy as jnp
from jax import lax
from jax.experimental import pallas as pl
from jax.experimental.pallas import tpu as pltpu

NZ = 4
CW, CCW = 0, 1
MESH = pl.DeviceIdType.MESH


def kernel(x):
    m, n = x.shape
    q = m // 4
    p = q // 2

    def body(init_ref, out_ref, zsend, zrecv, psend, precv):
        X = lax.axis_index("x")
        Y = lax.axis_index("y")
        Z = lax.axis_index("z")
        r = 2 * X + jnp.bitwise_xor(X, Y)

        def mod4(v):
            return lax.rem(v + 8, NZ)

        def pos_coords(pos):
            pos = mod4(pos)
            px = pos // 2
            return px, jnp.bitwise_xor(lax.rem(pos, 2), px)

        cwx, cwy = pos_coords(r + 1)
        ccwx, ccwy = pos_coords(r - 1)
        cw_tgt = (cwx, cwy, Z)
        ccw_tgt = (ccwx, ccwy, Z)
        zup = (X, Y, mod4(Z + 1))
        zdn = (X, Y, mod4(Z - 1))

        barrier = pltpu.get_barrier_semaphore()
        for tgt in (zup, zdn, cw_tgt, ccw_tgt):
            pl.semaphore_signal(barrier, inc=1, device_id=tgt,
                                device_id_type=MESH)
        pl.semaphore_wait(barrier, 4)

        def rows(o, start, size):
            return out_ref.at[pl.ds(o * m + start, size), :]

        pending = []

        def send(src, dst, sem_s, sem_r, tgt, cond):
            rdma = pltpu.make_async_remote_copy(
                src_ref=src, dst_ref=dst, send_sem=sem_s, recv_sem=sem_r,
                device_id=tgt, device_id_type=MESH)

            @pl.when(cond)
            def _():
                rdma.start()

            pending.append((cond, rdma))

        def wait_recv(dst, sem_s, sem_r, cond):
            rdma = pltpu.make_async_remote_copy(
                src_ref=dst, dst_ref=dst, send_sem=sem_s, recv_sem=sem_r,
                device_id=(X, Y, Z), device_id_type=MESH)

            @pl.when(cond)
            def _():
                rdma.wait_recv()

        def slot(flavor, h):
            if flavor == "R":
                return dict(o=mod4(Z - 1 - h), cond=(Z >= h + 1), j=2 - h)
            return dict(o=mod4(Z + 1 + h), cond=(Z <= 2 - h), j=h)

        def plane_hop0(s, half):
            o, j, cond = s["o"], s["j"], s["cond"]
            if half == 0:
                send(rows(o, r * q, p), rows(o, r * q, p),
                     psend.at[j, CW, 0], precv.at[j, CW, 0], cw_tgt, cond)
            else:
                send(rows(o, r * q + p, p), rows(o, r * q + p, p),
                     psend.at[j, CCW, 0], precv.at[j, CCW, 0], ccw_tgt,
                     cond)

        def plane_step(s, k):
            o, j, cond = s["o"], s["j"], s["cond"]
            rc = mod4(r - 1 - k)
            wait_recv(rows(o, rc * q, p),
                      psend.at[j, CW, k], precv.at[j, CW, k], cond)
            if k < 2:
                send(rows(o, rc * q, p), rows(o, rc * q, p),
                     psend.at[j, CW, k + 1], precv.at[j, CW, k + 1],
                     cw_tgt, cond)
            rl = mod4(r + 1 + k)
            wait_recv(rows(o, rl * q + p, p),
                      psend.at[j, CCW, k], precv.at[j, CCW, k], cond)
            if k < 2:
                send(rows(o, rl * q + p, p), rows(o, rl * q + p, p),
                     psend.at[j, CCW, k + 1], precv.at[j, CCW, k + 1],
                     ccw_tgt, cond)

        for half in (0, 1):
            src = rows(Z, r * q + half * p, p)
            send(src, src, zsend.at[0, 0, half], zrecv.at[0, 0, half],
                 zup, Z <= 2)
            send(src, src, zsend.at[1, 0, half], zrecv.at[1, 0, half],
                 zdn, Z >= 1)

        for t in range(6):
            for h in range(3):
                k = t - 1 - h
                if 0 <= k <= 2:
                    plane_step(slot("R", h), k)
                    plane_step(slot("L", h), k)
            if t < 3:
                for flavor, ztgt, f in (("R", zup, 0), ("L", zdn, 1)):
                    s = slot(flavor, t)
                    fwd_ok = s["cond"] & (
                        (Z <= 2) if flavor == "R" else (Z >= 1))
                    for half in (0, 1):
                        piece = rows(s["o"], r * q + half * p, p)
                        wait_recv(piece, zsend.at[f, t, half],
                                  zrecv.at[f, t, half], s["cond"])
                        if t < 2:
                            send(piece, piece,
                                 zsend.at[f, t + 1, half],
                                 zrecv.at[f, t + 1, half], ztgt, fwd_ok)
                        plane_hop0(s, half)

        for cond, rdma in pending:
            @pl.when(cond)
            def _(rdma=rdma):
                rdma.wait_send()

    out_init = jnp.tile(x.astype(jnp.bfloat16), (NZ, 1))
    return pl.pallas_call(
        body,
        out_shape=jax.ShapeDtypeStruct((NZ * m, n), jnp.bfloat16),
        in_specs=[pl.BlockSpec(memory_space=pl.ANY)],
        out_specs=pl.BlockSpec(memory_space=pl.ANY),
        input_output_aliases={0: 0},
        scratch_shapes=[
            pltpu.SemaphoreType.DMA((2, 3, 2)),
            pltpu.SemaphoreType.DMA((2, 3, 2)),
            pltpu.SemaphoreType.DMA((3, 2, 3)),
            pltpu.SemaphoreType.DMA((3, 2, 3)),
        ],
        compiler_params=pltpu.CompilerParams(collective_id=0),
    )(out_init)
